# baseline (speedup 1.0000x reference)
import functools

import jax
import jax.numpy as jnp
from jax import lax
from jax.experimental import pallas as pl
from jax.experimental.pallas import tpu as pltpu
from jax.experimental.pallas import tpu_sc as plsc

_NC = 2
_NS = 16
_NW = _NC * _NS


def _sdiff_body(tab_ref, wd_ref, bd_ref, s_ref):
    x = tab_ref[...]
    w = wd_ref[...]
    s_ref[...] = jnp.sum(x * w, axis=0) + bd_ref[0, 0]


@functools.lru_cache(maxsize=None)
def _make_sdiff(V, D, blk):
    return pl.pallas_call(
        _sdiff_body,
        grid=(pl.cdiv(V, blk),),
        in_specs=[
            pl.BlockSpec((D, blk), lambda i: (0, i)),
            pl.BlockSpec((D, 1), lambda i: (0, 0)),
            pl.BlockSpec(memory_space=pltpu.SMEM),
        ],
        out_specs=pl.BlockSpec((blk,), lambda i: (i,)),
        out_shape=jax.ShapeDtypeStruct((V,), jnp.float32),
    )


@functools.lru_cache(maxsize=None)
def _make_gather(V, N, CH, NJ):
    NB = N // _NW
    NCH = NB // CH
    SPC = CH // 1024
    mesh = plsc.VectorSubcoreMesh(core_axis_name="c", subcore_axis_name="s")

    @functools.partial(
        pl.kernel,
        out_type=jax.ShapeDtypeStruct((N,), jnp.float32),
        mesh=mesh,
        compiler_params=pltpu.CompilerParams(needs_layout_passes=False),
        scratch_types=[
            pltpu.VMEM((V,), jnp.float32),
            pltpu.VMEM((CH,), jnp.int32),
            pltpu.VMEM((1024,), jnp.float32),
            pltpu.VMEM((1024,), jnp.float32),
            pltpu.SemaphoreType.DMA,
            pltpu.SemaphoreType.DMA,
        ],
    )
    def gather_k(s_hbm, idx_hbm, g_hbm, s_v, idx_v, g0_v, g1_v, sem0, sem1):
        wid = lax.axis_index("s") * _NC + lax.axis_index("c")
        base = wid * NB
        pltpu.sync_copy(s_hbm, s_v)
        g_bufs = (g0_v, g1_v)
        sems = (sem0, sem1)

        def chunk(c, carry):
            pltpu.sync_copy(idx_hbm.at[pl.ds(base + c * CH, CH)], idx_v)
            for sub in range(SPC):
                par = sub % 2
                g_v, sem = g_bufs[par], sems[par]
                scc = c * SPC + sub
                G0 = wid * (NB // 128) + scc * 8
                lt = G0 // (8 * NJ)
                bt = (G0 // 8) % NJ
                obase = lt * (1024 * NJ) + bt * 128

                @pl.when(scc >= 2)
                def _drain():
                    for lp in range(8):
                        pltpu.make_async_copy(
                            g_v.at[pl.ds(lp * 128, 128)],
                            g_hbm.at[pl.ds(lp * 128, 128)], sem).wait()

                @plsc.parallel_loop(0, 8, unroll=4)
                def vec(lp):
                    for p in range(8):
                        iv = idx_v[pl.ds(sub * 1024 + lp * 128 + p * 16, 16)]
                        vals = plsc.load_gather(s_v, [iv])
                        g_v[pl.ds(lp * 128 + p * 16, 16)] = vals

                for lp in range(8):
                    pltpu.async_copy(
                        g_v.at[pl.ds(lp * 128, 128)],
                        g_hbm.at[pl.ds(obase + lp * (128 * NJ), 128)],
                        sem)
            return carry

        lax.fori_loop(0, NCH, chunk, 0)
        for par in range(2):
            g_v, sem = g_bufs[par], sems[par]
            for lp in range(8):
                pltpu.make_async_copy(
                    g_v.at[pl.ds(lp * 128, 128)],
                    g_hbm.at[pl.ds(lp * 128, 128)], sem).wait()

    return gather_k


def _softplus_body(g_ref, o_ref):
    g = g_ref[...]
    u1 = -(jnp.maximum(g, 0.0) + jnp.log1p(jnp.exp(-jnp.abs(g))))
    blk = g.shape[0]
    o_ref[...] = jnp.stack([u1 + g, u1], axis=1).reshape(2 * blk, 128)


@functools.lru_cache(maxsize=None)
def _make_softplus(R, blk):
    return pl.pallas_call(
        _softplus_body,
        grid=(R // blk,),
        in_specs=[pl.BlockSpec((blk, 128), lambda i: (i, 0))],
        out_specs=pl.BlockSpec((2 * blk, 128), lambda i: (i, 0)),
        out_shape=jax.ShapeDtypeStruct((2 * R, 128), jnp.float32),
    )


def kernel(input, table, W, b):
    B, L = input.shape
    V, D = table.shape
    N = B * L
    NJ = B // 128

    wd = (W[0] - W[1]).reshape(D, 1).astype(jnp.float32)
    bd = (b[0] - b[1]).reshape(1, 1).astype(jnp.float32)

    LT = L // 8
    s = _make_sdiff(V, D, 8192)(table.T, wd, bd)
    idx_raw = (input.T.reshape(LT, 8, NJ, 128).transpose(0, 2, 1, 3)
               .reshape(N).astype(jnp.int32))
    g1 = _make_gather(V, N, 5120, NJ)(s, idx_raw)
    o = _make_softplus(N // 128, 800)(g1.reshape(N // 128, 128))
    return (o.reshape(L, NJ, 2, 128).transpose(1, 3, 0, 2).reshape(B, L, 2))
